# baseline (speedup 1.0000x reference)
import jax
import jax.numpy as jnp
from jax.experimental import pallas as pl

NUM_AGENTS = 16384
NUM_ABS = 64
LANES = 128
R = 1024
ROWS2 = R * NUM_ABS // LANES


def _rotl(x, r):
    return (x << jnp.uint32(r)) | (x >> jnp.uint32(32 - r))


def _threefry_bits(cnt):
    ks0 = jnp.uint32(0)
    ks1 = jnp.uint32(42)
    ks2 = jnp.uint32(0x1BD11BDA ^ 42)

    def rounds(x0, x1, rots):
        for r in rots:
            x0 = x0 + x1
            x1 = _rotl(x1, r)
            x1 = x0 ^ x1
        return x0, x1

    x0 = jnp.zeros_like(cnt)
    x1 = cnt + ks1
    x0, x1 = rounds(x0, x1, (13, 15, 26, 6))
    x0, x1 = x0 + ks1, x1 + ks2 + jnp.uint32(1)
    x0, x1 = rounds(x0, x1, (17, 29, 16, 24))
    x0, x1 = x0 + ks2, x1 + ks0 + jnp.uint32(2)
    x0, x1 = rounds(x0, x1, (13, 15, 26, 6))
    x0, x1 = x0 + ks0, x1 + ks1 + jnp.uint32(3)
    x0, x1 = rounds(x0, x1, (17, 29, 16, 24))
    x0, x1 = x0 + ks1, x1 + ks2 + jnp.uint32(4)
    x0, x1 = rounds(x0, x1, (13, 15, 26, 6))
    x0, x1 = x0 + ks2, x1 + ks0 + jnp.uint32(5)
    return x0 ^ x1


def _body(logits_ref, out_ref):
    i = pl.program_id(0)
    b = pl.program_id(1)
    base = (b * (NUM_AGENTS * NUM_ABS) + i * (R * NUM_ABS)).astype(jnp.uint32)
    row = jax.lax.broadcasted_iota(jnp.uint32, (ROWS2, LANES), 0)
    lane = jax.lax.broadcasted_iota(jnp.uint32, (ROWS2, LANES), 1)
    cnt = base + row * jnp.uint32(LANES) + lane

    bits = _threefry_bits(cnt)
    fb = (bits >> jnp.uint32(9)) | jnp.uint32(0x3F800000)
    f = jax.lax.bitcast_convert_type(fb, jnp.float32) - jnp.float32(1.0)
    u = jnp.maximum(jnp.float32(1e-20), f + jnp.float32(1e-20))
    g = -jnp.log(-jnp.log(u))
    z = logits_ref[...] + g

    ka = jax.lax.broadcasted_iota(jnp.int32, (ROWS2, NUM_ABS), 1)
    za = z[:, :NUM_ABS]
    zb = z[:, NUM_ABS:]
    ma = jnp.max(za, axis=1, keepdims=True)
    mb = jnp.max(zb, axis=1, keepdims=True)
    ia = jnp.min(jnp.where(za == ma, ka, NUM_ABS), axis=1)
    ib = jnp.min(jnp.where(zb == mb, ka, NUM_ABS), axis=1)
    out_ref[0, 0, :] = ia
    out_ref[0, 1, :] = ib


def _run(batch):
    nblk = NUM_AGENTS // R
    return pl.pallas_call(
        _body,
        grid=(nblk, batch),
        in_specs=[pl.BlockSpec((ROWS2, LANES), lambda i, b: (i, 0))],
        out_specs=pl.BlockSpec((1, 2, ROWS2), lambda i, b: (b, 0, i)),
        out_shape=jax.ShapeDtypeStruct((batch, 2, NUM_AGENTS // 2), jnp.int32),
    )


def kernel(state, assigner_logit_array):
    logits_flat = assigner_logit_array.reshape(
        NUM_AGENTS * NUM_ABS // LANES, LANES)
    if state.ndim == 2:
        batch = state.shape[0]
        out3 = _run(batch)(logits_flat)
        return jnp.moveaxis(out3, 1, 2).reshape(batch, NUM_AGENTS)
    out3 = _run(1)(logits_flat)
    return jnp.moveaxis(out3, 1, 2).reshape(NUM_AGENTS)


# speedup vs baseline: 1.4477x; 1.4477x over previous
import jax
import jax.numpy as jnp
from jax.experimental import pallas as pl

NUM_AGENTS = 16384
NUM_ABS = 64
C = 2048
NBLK = NUM_AGENTS // C


def _rotl(x, r):
    return (x << jnp.uint32(r)) | (x >> jnp.uint32(32 - r))


def _threefry_bits(cnt):
    ks0 = jnp.uint32(0)
    ks1 = jnp.uint32(42)
    ks2 = jnp.uint32(0x1BD11BDA ^ 42)

    def rounds(x0, x1, rots):
        for r in rots:
            x0 = x0 + x1
            x1 = _rotl(x1, r)
            x1 = x0 ^ x1
        return x0, x1

    x0 = jnp.zeros_like(cnt)
    x1 = cnt + ks1
    x0, x1 = rounds(x0, x1, (13, 15, 26, 6))
    x0, x1 = x0 + ks1, x1 + ks2 + jnp.uint32(1)
    x0, x1 = rounds(x0, x1, (17, 29, 16, 24))
    x0, x1 = x0 + ks2, x1 + ks0 + jnp.uint32(2)
    x0, x1 = rounds(x0, x1, (13, 15, 26, 6))
    x0, x1 = x0 + ks0, x1 + ks1 + jnp.uint32(3)
    x0, x1 = rounds(x0, x1, (17, 29, 16, 24))
    x0, x1 = x0 + ks1, x1 + ks2 + jnp.uint32(4)
    x0, x1 = rounds(x0, x1, (13, 15, 26, 6))
    x0, x1 = x0 + ks2, x1 + ks0 + jnp.uint32(5)
    return x0 ^ x1


def _body(logits_ref, out_ref):
    i = pl.program_id(0)
    b = pl.program_id(1)
    base = (b * (NUM_AGENTS * NUM_ABS) + i * (C * NUM_ABS)).astype(jnp.uint32)
    krow = jax.lax.broadcasted_iota(jnp.uint32, (NUM_ABS, C), 0)
    ncol = jax.lax.broadcasted_iota(jnp.uint32, (NUM_ABS, C), 1)
    cnt = base + ncol * jnp.uint32(NUM_ABS) + krow

    bits = _threefry_bits(cnt)
    fb = (bits >> jnp.uint32(9)) | jnp.uint32(0x3F800000)
    f = jax.lax.bitcast_convert_type(fb, jnp.float32) - jnp.float32(1.0)
    u = jnp.maximum(jnp.float32(1e-20), f + jnp.float32(1e-20))
    g = -jnp.log(-jnp.log(u))
    z = logits_ref[...] + g

    m = jnp.max(z, axis=0, keepdims=True)
    idx = jnp.min(jnp.where(z == m, krow.astype(jnp.int32), NUM_ABS), axis=0)
    out_ref[0, 0, :] = idx


def _run(batch):
    return pl.pallas_call(
        _body,
        grid=(NBLK, batch),
        in_specs=[pl.BlockSpec((NUM_ABS, C), lambda i, b: (0, i))],
        out_specs=pl.BlockSpec((1, 1, C), lambda i, b: (b * NBLK + i, 0, 0)),
        out_shape=jax.ShapeDtypeStruct((batch * NBLK, 1, C), jnp.int32),
    )


def kernel(state, assigner_logit_array):
    logits_t = assigner_logit_array.T
    if state.ndim == 2:
        batch = state.shape[0]
        return _run(batch)(logits_t).reshape(batch, NUM_AGENTS)
    return _run(1)(logits_t).reshape(NUM_AGENTS)


# speedup vs baseline: 1.8458x; 1.2750x over previous
import jax
import jax.numpy as jnp
from jax.experimental import pallas as pl

NUM_AGENTS = 16384
NUM_ABS = 64
C = 2048
NBLK = NUM_AGENTS // C


def _rotl(x, r):
    return (x << jnp.uint32(r)) | (x >> jnp.uint32(32 - r))


def _threefry_bits(cnt):
    ks0 = jnp.uint32(0)
    ks1 = jnp.uint32(42)
    ks2 = jnp.uint32(0x1BD11BDA ^ 42)

    def rounds(x0, x1, rots):
        for r in rots:
            x0 = x0 + x1
            x1 = _rotl(x1, r)
            x1 = x0 ^ x1
        return x0, x1

    x0 = jnp.zeros_like(cnt)
    x1 = cnt + ks1
    x0, x1 = rounds(x0, x1, (13, 15, 26, 6))
    x0, x1 = x0 + ks1, x1 + ks2 + jnp.uint32(1)
    x0, x1 = rounds(x0, x1, (17, 29, 16, 24))
    x0, x1 = x0 + ks2, x1 + ks0 + jnp.uint32(2)
    x0, x1 = rounds(x0, x1, (13, 15, 26, 6))
    x0, x1 = x0 + ks0, x1 + ks1 + jnp.uint32(3)
    x0, x1 = rounds(x0, x1, (17, 29, 16, 24))
    x0, x1 = x0 + ks1, x1 + ks2 + jnp.uint32(4)
    x0, x1 = rounds(x0, x1, (13, 15, 26, 6))
    x0, x1 = x0 + ks2, x1 + ks0 + jnp.uint32(5)
    return x0 ^ x1


TILE = 256


def _body(logits_ref, out_ref):
    i = pl.program_id(0)
    b = pl.program_id(1)
    base = (b * (NUM_AGENTS * NUM_ABS) + i * (C * NUM_ABS)).astype(jnp.uint32)
    krow = jax.lax.broadcasted_iota(jnp.uint32, (NUM_ABS, TILE), 0)
    ncol = jax.lax.broadcasted_iota(jnp.uint32, (NUM_ABS, TILE), 1)
    kidx = krow.astype(jnp.int32)

    def tile_step(j, carry):
        col0 = j * TILE
        cnt = (base + col0.astype(jnp.uint32) * jnp.uint32(NUM_ABS)
               + ncol * jnp.uint32(NUM_ABS) + krow)
        bits = _threefry_bits(cnt)
        fb = (bits >> jnp.uint32(9)) | jnp.uint32(0x3F800000)
        f = jax.lax.bitcast_convert_type(fb, jnp.float32) - jnp.float32(1.0)
        u = jnp.maximum(jnp.float32(1e-20), f + jnp.float32(1e-20))
        g = -jnp.log(-jnp.log(u))
        z = logits_ref[:, pl.ds(col0, TILE)] + g

        m = jnp.max(z, axis=0, keepdims=True)
        idx = jnp.min(jnp.where(z == m, kidx, NUM_ABS), axis=0)
        out_ref[0, 0, pl.ds(col0, TILE)] = idx
        return carry

    jax.lax.fori_loop(0, C // TILE, tile_step, 0)


def _run(batch):
    return pl.pallas_call(
        _body,
        grid=(NBLK, batch),
        in_specs=[pl.BlockSpec((NUM_ABS, C), lambda i, b: (0, i))],
        out_specs=pl.BlockSpec((1, 1, C), lambda i, b: (b * NBLK + i, 0, 0)),
        out_shape=jax.ShapeDtypeStruct((batch * NBLK, 1, C), jnp.int32),
    )


def kernel(state, assigner_logit_array):
    logits_t = assigner_logit_array.T
    if state.ndim == 2:
        batch = state.shape[0]
        return _run(batch)(logits_t).reshape(batch, NUM_AGENTS)
    return _run(1)(logits_t).reshape(NUM_AGENTS)


# speedup vs baseline: 1.9731x; 1.0690x over previous
import jax
import jax.numpy as jnp
from jax.experimental import pallas as pl

NUM_AGENTS = 16384
NUM_ABS = 64
C = 2048
NBLK = NUM_AGENTS // C


def _rotl(x, r):
    return (x << jnp.uint32(r)) | (x >> jnp.uint32(32 - r))


def _threefry_bits(x1):
    ks0 = jnp.uint32(0)
    ks1 = jnp.uint32(42)
    ks2 = jnp.uint32(0x1BD11BDA ^ 42)

    def rounds(x0, x1, rots):
        for r in rots:
            x0 = x0 + x1
            x1 = _rotl(x1, r)
            x1 = x0 ^ x1
        return x0, x1

    x0 = jnp.zeros_like(x1)
    x0, x1 = rounds(x0, x1, (13, 15, 26, 6))
    x0, x1 = x0 + ks1, x1 + ks2 + jnp.uint32(1)
    x0, x1 = rounds(x0, x1, (17, 29, 16, 24))
    x0, x1 = x0 + ks2, x1 + ks0 + jnp.uint32(2)
    x0, x1 = rounds(x0, x1, (13, 15, 26, 6))
    x0, x1 = x0 + ks0, x1 + ks1 + jnp.uint32(3)
    x0, x1 = rounds(x0, x1, (17, 29, 16, 24))
    x0, x1 = x0 + ks1, x1 + ks2 + jnp.uint32(4)
    x0, x1 = rounds(x0, x1, (13, 15, 26, 6))
    x0, x1 = x0 + ks2, x1 + ks0 + jnp.uint32(5)
    return x0 ^ x1


TILE = 256


def _body(logits_ref, out_ref):
    i = pl.program_id(0)
    b = pl.program_id(1)
    base = (b * (NUM_AGENTS * NUM_ABS) + i * (C * NUM_ABS)).astype(jnp.uint32)
    krow = jax.lax.broadcasted_iota(jnp.uint32, (NUM_ABS, TILE), 0)
    ncol = jax.lax.broadcasted_iota(jnp.uint32, (NUM_ABS, TILE), 1)
    cnt_lo = ncol * jnp.uint32(NUM_ABS) + krow
    kidx = krow.astype(jnp.int32)

    for j in range(C // TILE):
        col0 = j * TILE
        x1 = cnt_lo + (base + jnp.uint32(col0 * NUM_ABS + 42))
        bits = _threefry_bits(x1)
        fb = (bits >> jnp.uint32(9)) | jnp.uint32(0x3F800000)
        f = jax.lax.bitcast_convert_type(fb, jnp.float32) - jnp.float32(1.0)
        u = jnp.maximum(jnp.float32(1e-20), f + jnp.float32(1e-20))
        g = -jnp.log(-jnp.log(u))
        z = logits_ref[:, pl.ds(col0, TILE)] + g

        m = jnp.max(z, axis=0, keepdims=True)
        idx = jnp.min(jnp.where(z == m, kidx, NUM_ABS), axis=0)
        out_ref[0, 0, pl.ds(col0, TILE)] = idx


def _run(batch):
    return pl.pallas_call(
        _body,
        grid=(NBLK, batch),
        in_specs=[pl.BlockSpec((NUM_ABS, C), lambda i, b: (0, i))],
        out_specs=pl.BlockSpec((1, 1, C), lambda i, b: (b * NBLK + i, 0, 0)),
        out_shape=jax.ShapeDtypeStruct((batch * NBLK, 1, C), jnp.int32),
    )


def kernel(state, assigner_logit_array):
    logits_t = assigner_logit_array.T
    if state.ndim == 2:
        batch = state.shape[0]
        return _run(batch)(logits_t).reshape(batch, NUM_AGENTS)
    return _run(1)(logits_t).reshape(NUM_AGENTS)


# speedup vs baseline: 2.0745x; 1.0513x over previous
import jax
import jax.numpy as jnp
from jax.experimental import pallas as pl

NUM_AGENTS = 16384
NUM_ABS = 64
C = 2048
NBLK = NUM_AGENTS // C


def _rotl(x, r):
    return (x << jnp.uint32(r)) | (x >> jnp.uint32(32 - r))


def _threefry_bits(x1):
    ks0 = jnp.uint32(0)
    ks1 = jnp.uint32(42)
    ks2 = jnp.uint32(0x1BD11BDA ^ 42)

    def rounds(x0, x1, rots):
        for r in rots:
            x0 = x0 + x1
            x1 = _rotl(x1, r)
            x1 = x0 ^ x1
        return x0, x1

    x0 = jnp.zeros_like(x1)
    x0, x1 = rounds(x0, x1, (13, 15, 26, 6))
    x0, x1 = x0 + ks1, x1 + ks2 + jnp.uint32(1)
    x0, x1 = rounds(x0, x1, (17, 29, 16, 24))
    x0, x1 = x0 + ks2, x1 + ks0 + jnp.uint32(2)
    x0, x1 = rounds(x0, x1, (13, 15, 26, 6))
    x0, x1 = x0 + ks0, x1 + ks1 + jnp.uint32(3)
    x0, x1 = rounds(x0, x1, (17, 29, 16, 24))
    x0, x1 = x0 + ks1, x1 + ks2 + jnp.uint32(4)
    x0, x1 = rounds(x0, x1, (13, 15, 26, 6))
    x0, x1 = x0 + ks2, x1 + ks0 + jnp.uint32(5)
    return x0 ^ x1


TILE = 256


def _body(logits_ref, out_ref):
    i = pl.program_id(0)
    b = pl.program_id(1)
    base = (b * (NUM_AGENTS * NUM_ABS) + i * (C * NUM_ABS)).astype(jnp.uint32)
    krow = jax.lax.broadcasted_iota(jnp.uint32, (NUM_ABS, TILE), 0)
    ncol = jax.lax.broadcasted_iota(jnp.uint32, (NUM_ABS, TILE), 1)
    cnt_lo = ncol * jnp.uint32(NUM_ABS) + krow
    kidx = krow.astype(jnp.int32)

    for j in range(C // TILE):
        col0 = j * TILE
        x1 = cnt_lo + (base + jnp.uint32(col0 * NUM_ABS + 42))
        bits = _threefry_bits(x1)
        m_bits = bits >> jnp.uint32(9)
        mf = jax.lax.bitcast_convert_type(m_bits, jnp.int32).astype(jnp.float32)
        s = jnp.log2(mf)
        q = (s - jnp.float32(23.0)) * logits_ref[:, pl.ds(col0, TILE)]

        m = jnp.max(q, axis=0, keepdims=True)
        idx = jnp.min(jnp.where(q == m, kidx, NUM_ABS), axis=0)
        out_ref[0, 0, pl.ds(col0, TILE)] = idx


def _run(batch):
    return pl.pallas_call(
        _body,
        grid=(NBLK, batch),
        in_specs=[pl.BlockSpec((NUM_ABS, C), lambda i, b: (0, i))],
        out_specs=pl.BlockSpec((1, 1, C), lambda i, b: (b * NBLK + i, 0, 0)),
        out_shape=jax.ShapeDtypeStruct((batch * NBLK, 1, C), jnp.int32),
    )


def kernel(state, assigner_logit_array):
    w_t = (jnp.float32(0.6931471805599453)
           * jnp.exp(-assigner_logit_array)).T
    if state.ndim == 2:
        batch = state.shape[0]
        return _run(batch)(w_t).reshape(batch, NUM_AGENTS)
    return _run(1)(w_t).reshape(NUM_AGENTS)
